# baseline (speedup 1.0000x reference)
import numpy as np

import jax
import jax.numpy as jnp
from jax.experimental import pallas as pl
from jax.experimental.pallas import tpu as pltpu

_D = 263
_B = 4
_T0 = 64
_WIDTH = 512
_CODE_DIM = 32
_NB_CODE = 256
_DEPTH = 3
_DOWN_T = 3
_DGR = 3

_MEAN_UPPER = np.asarray([0.1216, 0.2488, 0.2967, 0.5027, 0.4053, 0.41,
                          0.5703, 0.403, 0.4078, 0.1994, 0.1992, 0.0661,
                          0.0639], dtype=np.float32)
_STD_UPPER = np.asarray([0.0164, 0.0412, 0.0523, 0.0864, 0.0695, 0.0703,
                         0.1108, 0.0853, 0.0847, 0.1289, 0.1291, 0.2463,
                         0.2484], dtype=np.float32)
_SPINE_IDX = np.arange(0, 60)
_LA_IDX = np.arange(60, 108)
_RA_IDX = np.arange(101, 149)
_LL_IDX = np.arange(149, 208)
_RL_IDX = np.concatenate([np.arange(149, 153), np.arange(208, 263)])
_LOWER_MAP = np.array([0, 1, 2, 3])
_OVERLAP_LOWER_IDX = np.arange(149, 153)
_UPPER_Y_IDX = np.array([60 + 4 * i for i in range(13)])

_PARTS = ("left_arm", "right_arm", "right_leg", "left_leg", "spine")
_PART_IDX = {"left_arm": _LA_IDX, "right_arm": _RA_IDX, "right_leg": _RL_IDX,
             "left_leg": _LL_IDX, "spine": _SPINE_IDX}
_PART_DIM = {"left_arm": 48, "right_arm": 48, "right_leg": 59,
             "left_leg": 59, "spine": 60}
_DPAD = 64


def _shift(x3, s):
    b, t, c = x3.shape
    if s == 0:
        return x3
    z = jnp.zeros((b, abs(s), c), dtype=x3.dtype)
    if s > 0:
        return jnp.concatenate([x3[:, s:, :], z], axis=1)
    return jnp.concatenate([z, x3[:, :s, :]], axis=1)


def _mm(a2, w2):
    return jax.lax.dot_general(a2, w2, (((1,), (0,)), ((), ())),
                               preferred_element_type=jnp.float32)


def _conv_taps(x3, shifts, ws, bias):
    b, t, _ = x3.shape
    acc = None
    for s, w in zip(shifts, ws):
        if abs(s) >= t:
            continue
        y = _mm(_shift(x3, s).reshape(b * t, -1), w)
        acc = y if acc is None else acc + y
    out = acc.reshape(b, t, -1)
    if bias is not None:
        out = out + bias[None, None, :]
    return out


def _down_conv(x3, ws, bias):
    b, t, c = x3.shape
    to = t // 2
    acc = None
    for k in range(4):
        xs = _shift(x3, k - 1).reshape(b, to, 2, c)[:, :, 0, :]
        y = _mm(xs.reshape(b * to, c), ws[k])
        acc = y if acc is None else acc + y
    return acc.reshape(b, to, -1) + bias[None, None, :]


def _res_block(x3, w1s, b1, w2, b2, d):
    h = jax.nn.relu(x3)
    h = _conv_taps(h, (-d, 0, d), w1s, b1)
    h = jax.nn.relu(h)
    h = _conv_taps(h, (0,), (w2,), b2)
    return x3 + h


def _enc_level(xv, wlev_ref, blev_ref):
    y = _down_conv(xv, [wlev_ref[0, 0, k] for k in range(4)], blev_ref[0, 0, 0])
    for j in range(_DEPTH):
        d = _DGR ** j
        base = 4 + 4 * j
        y = _res_block(y,
                       [wlev_ref[0, 0, base + k] for k in range(3)],
                       blev_ref[0, 0, 1 + 2 * j],
                       wlev_ref[0, 0, base + 3],
                       blev_ref[0, 0, 2 + 2 * j], d)
    return y


def _enc_kernel(xin_ref, win_ref, bin_ref, wlev_ref, blev_ref, wout_ref,
                bout_ref, cb_ref, cbt_ref, q_ref, stats_ref, xs_ref):
    lvl = pl.program_id(1)

    @pl.when(lvl == 0)
    def _l0():
        xv = xin_ref[0]
        h = jax.nn.relu(_conv_taps(xv, (-1, 0, 1),
                                   [win_ref[0, k] for k in range(3)],
                                   bin_ref[0, 0]))
        y = _enc_level(h, wlev_ref, blev_ref)
        xs_ref[:, :32, :] = y

    @pl.when(lvl == 1)
    def _l1():
        y = _enc_level(xs_ref[:, :32, :], wlev_ref, blev_ref)
        xs_ref[:, :16, :] = y

    @pl.when(lvl == 2)
    def _l2():
        y = _enc_level(xs_ref[:, :16, :], wlev_ref, blev_ref)
        e = _conv_taps(y, (-1, 0, 1), [wout_ref[0, k] for k in range(3)],
                       bout_ref[0, 0])
        n = _B * 8
        xf = e.reshape(n, _CODE_DIM)
        cb = cb_ref[0]
        dist = (jnp.sum(xf * xf, axis=1, keepdims=True)
                - 2.0 * _mm(xf, cbt_ref[0])
                + jnp.sum(cb * cb, axis=1)[None, :])
        dmin = jnp.min(dist, axis=1, keepdims=True)
        lane = jax.lax.broadcasted_iota(jnp.int32, (n, _NB_CODE), 1)
        idx = jnp.min(jnp.where(dist <= dmin, lane, _NB_CODE), axis=1)
        onehot = (lane == idx[:, None]).astype(jnp.float32)
        xd = _mm(onehot, cb)
        loss = jnp.mean((xf - xd) ** 2)
        pr = jnp.mean(onehot, axis=0)
        perp = jnp.exp(-jnp.sum(pr * jnp.log(pr + 1e-10)))
        q_ref[0] = xd.reshape(_B, 8, _CODE_DIM)
        row = jax.lax.broadcasted_iota(jnp.int32, (8, 128), 0)
        stats = jnp.where(row == 0, loss, jnp.where(row == 1, perp, 0.0))
        stats_ref[0] = stats.astype(jnp.float32)


def _dec_level(xv, wlev_ref, blev_ref):
    for j in range(_DEPTH):
        d = _DGR ** (_DEPTH - 1 - j)
        base = 4 * j
        xv = _res_block(xv,
                        [wlev_ref[0, 0, base + k] for k in range(3)],
                        blev_ref[0, 0, 2 * j],
                        wlev_ref[0, 0, base + 3],
                        blev_ref[0, 0, 2 * j + 1], d)
    b, t, c = xv.shape
    xr = jnp.broadcast_to(xv[:, :, None, :], (b, t, 2, c)).reshape(b, 2 * t, c)
    return _conv_taps(xr, (-1, 0, 1), [wlev_ref[0, 0, 12 + k] for k in range(3)],
                      blev_ref[0, 0, 6])


def _dec_kernel(q_ref, win_ref, bin_ref, wlev_ref, blev_ref, wmid_ref,
                bmid_ref, wout_ref, bout_ref, y_ref, xs_ref):
    lvl = pl.program_id(1)

    @pl.when(lvl == 0)
    def _l0():
        q = q_ref[0]
        h = jax.nn.relu(_conv_taps(q, (-1, 0, 1),
                                   [win_ref[0, k] for k in range(3)],
                                   bin_ref[0, 0]))
        y = _dec_level(h, wlev_ref, blev_ref)
        xs_ref[:, :16, :] = y

    @pl.when(lvl == 1)
    def _l1():
        y = _dec_level(xs_ref[:, :16, :], wlev_ref, blev_ref)
        xs_ref[:, :32, :] = y

    @pl.when(lvl == 2)
    def _l2():
        y = _dec_level(xs_ref[:, :32, :], wlev_ref, blev_ref)
        h = jax.nn.relu(_conv_taps(y, (-1, 0, 1),
                                   [wmid_ref[0, k] for k in range(3)],
                                   bmid_ref[0, 0]))
        out = _conv_taps(h, (-1, 0, 1), [wout_ref[0, k] for k in range(3)],
                         bout_ref[0, 0])
        y_ref[0] = out


def _t(w):
    return jnp.transpose(w, (2, 1, 0))


def _pad_last(a, n):
    pad = n - a.shape[-1]
    if pad == 0:
        return a
    cfg = [(0, 0)] * (a.ndim - 1) + [(0, pad)]
    return jnp.pad(a, cfg)


def _pad_mid(a, n):
    pad = n - a.shape[1]
    if pad == 0:
        return a
    cfg = [(0, 0), (0, pad)] + [(0, 0)] * (a.ndim - 2)
    return jnp.pad(a, cfg)


def _pack_enc(params):
    win, bin_, wlev, blev, wout, bout, cb, cbt = [], [], [], [], [], [], [], []
    for name in _PARTS:
        p = params["enc"][name]
        win.append(_pad_mid(_t(p["w_in"]), _DPAD))
        bin_.append(p["b_in"][None, :])
        levw, levb = [], []
        for blk in p["down"]:
            taps = [_t(blk["w"])[k] for k in range(4)]
            bs = [blk["b"]]
            for rb in blk["res"]:
                taps += [_t(rb["w1"])[k] for k in range(3)]
                taps.append(_t(rb["w2"])[0])
                bs += [rb["b1"], rb["b2"]]
            bs.append(jnp.zeros((_WIDTH,), jnp.float32))
            levw.append(jnp.stack(taps))
            levb.append(jnp.stack(bs))
        wlev.append(jnp.stack(levw))
        blev.append(jnp.stack(levb))
        wout.append(_t(p["w_out"]))
        bout.append(p["b_out"][None, :])
        c = params["cb"][name]
        cb.append(c)
        cbt.append(c.T)
    return (jnp.stack(win), jnp.stack(bin_), jnp.stack(wlev),
            jnp.stack(blev), jnp.stack(wout), jnp.stack(bout),
            jnp.stack(cb), jnp.stack(cbt))


def _pack_dec(params):
    win, bin_, wlev, blev, wmid, bmid, wout, bout = [], [], [], [], [], [], [], []
    for name in _PARTS:
        p = params["dec"][name]
        win.append(_t(p["w_in"]))
        bin_.append(p["b_in"][None, :])
        levw, levb = [], []
        for blk in p["up"]:
            taps, bs = [], []
            for rb in blk["res"]:
                taps += [_t(rb["w1"])[k] for k in range(3)]
                taps.append(_t(rb["w2"])[0])
                bs += [rb["b1"], rb["b2"]]
            taps += [_t(blk["w"])[k] for k in range(3)]
            bs += [blk["b"], jnp.zeros((_WIDTH,), jnp.float32)]
            levw.append(jnp.stack(taps))
            levb.append(jnp.stack(bs))
        wlev.append(jnp.stack(levw))
        blev.append(jnp.stack(levb))
        wmid.append(_t(p["w_mid"]))
        bmid.append(p["b_mid"][None, :])
        wout.append(_pad_last(_t(p["w_out"]), _DPAD))
        bout.append(_pad_last(p["b_out"], _DPAD)[None, :])
    return (jnp.stack(win), jnp.stack(bin_), jnp.stack(wlev), jnp.stack(blev),
            jnp.stack(wmid), jnp.stack(bmid), jnp.stack(wout), jnp.stack(bout))


def _shift_upper_down(x):
    shift_y = x[:, :, 3:4]
    upper = (x[:, :, _UPPER_Y_IDX] - shift_y - _MEAN_UPPER) / _STD_UPPER
    return x.at[:, :, _UPPER_Y_IDX].set(upper)


def _shift_upper_up(x):
    upper = x[:, :, _UPPER_Y_IDX] * _STD_UPPER + _MEAN_UPPER
    x = x.at[:, :, _UPPER_Y_IDX].set(upper)
    shift_y = x[:, :, 3:4]
    return x.at[:, :, _UPPER_Y_IDX].add(shift_y)


def _merge(la, ra, rl, ll, sp):
    motion = jnp.zeros((_B, _T0, _D), dtype=la.dtype)
    motion = motion.at[:, :, _LA_IDX].set(la)
    motion = motion.at[:, :, _RA_IDX].set(ra)
    motion = motion.at[:, :, _RL_IDX].set(rl)
    motion = motion.at[:, :, _LL_IDX].set(ll)
    motion = motion.at[:, :, _SPINE_IDX].set(sp)
    return motion.at[:, :, _OVERLAP_LOWER_IDX].set(
        (ll[:, :, _LOWER_MAP] + rl[:, :, _LOWER_MAP]) / 2.0)


def kernel(x, params):
    x = x.astype(jnp.float32)
    xs = _shift_upper_down(x)
    xp = jnp.stack([_pad_last(xs[:, :, _PART_IDX[n]], _DPAD) for n in _PARTS])

    ew = _pack_enc(params)
    dw = _pack_dec(params)

    f32 = jnp.float32
    grid = (5, _DOWN_T)

    enc_specs = [
        pl.BlockSpec((1, _B, _T0, _DPAD), lambda p, l: (p, 0, 0, 0)),
        pl.BlockSpec((1, 3, _DPAD, _WIDTH), lambda p, l: (p, 0, 0, 0)),
        pl.BlockSpec((1, 1, _WIDTH), lambda p, l: (p, 0, 0)),
        pl.BlockSpec((1, 1, 16, _WIDTH, _WIDTH), lambda p, l: (p, l, 0, 0, 0)),
        pl.BlockSpec((1, 1, 8, _WIDTH), lambda p, l: (p, l, 0, 0)),
        pl.BlockSpec((1, 3, _WIDTH, _CODE_DIM), lambda p, l: (p, 0, 0, 0)),
        pl.BlockSpec((1, 1, _CODE_DIM), lambda p, l: (p, 0, 0)),
        pl.BlockSpec((1, _NB_CODE, _CODE_DIM), lambda p, l: (p, 0, 0)),
        pl.BlockSpec((1, _CODE_DIM, _NB_CODE), lambda p, l: (p, 0, 0)),
    ]
    q, stats = pl.pallas_call(
        _enc_kernel,
        grid=grid,
        in_specs=enc_specs,
        out_specs=[
            pl.BlockSpec((1, _B, 8, _CODE_DIM), lambda p, l: (p, 0, 0, 0)),
            pl.BlockSpec((1, 8, 128), lambda p, l: (p, 0, 0)),
        ],
        out_shape=[
            jax.ShapeDtypeStruct((5, _B, 8, _CODE_DIM), f32),
            jax.ShapeDtypeStruct((5, 8, 128), f32),
        ],
        scratch_shapes=[pltpu.VMEM((_B, _T0, _WIDTH), f32)],
    )(xp, *ew)

    dec_specs = [
        pl.BlockSpec((1, _B, 8, _CODE_DIM), lambda p, l: (p, 0, 0, 0)),
        pl.BlockSpec((1, 3, _CODE_DIM, _WIDTH), lambda p, l: (p, 0, 0, 0)),
        pl.BlockSpec((1, 1, _WIDTH), lambda p, l: (p, 0, 0)),
        pl.BlockSpec((1, 1, 15, _WIDTH, _WIDTH), lambda p, l: (p, l, 0, 0, 0)),
        pl.BlockSpec((1, 1, 8, _WIDTH), lambda p, l: (p, l, 0, 0)),
        pl.BlockSpec((1, 3, _WIDTH, _WIDTH), lambda p, l: (p, 0, 0, 0)),
        pl.BlockSpec((1, 1, _WIDTH), lambda p, l: (p, 0, 0)),
        pl.BlockSpec((1, 3, _WIDTH, _DPAD), lambda p, l: (p, 0, 0, 0)),
        pl.BlockSpec((1, 1, _DPAD), lambda p, l: (p, 0, 0)),
    ]
    y = pl.pallas_call(
        _dec_kernel,
        grid=grid,
        in_specs=dec_specs,
        out_specs=pl.BlockSpec((1, _B, _T0, _DPAD), lambda p, l: (p, 0, 0, 0)),
        out_shape=jax.ShapeDtypeStruct((5, _B, _T0, _DPAD), f32),
        scratch_shapes=[pltpu.VMEM((_B, _T0, _WIDTH), f32)],
    )(q, *dw)

    la = y[0][:, :, :48]
    ra = y[1][:, :, :48]
    rl = y[2][:, :, :59]
    ll = y[3][:, :, :59]
    sp = y[4][:, :, :60]
    motion = _shift_upper_up(_merge(la, ra, rl, ll, sp))
    loss = jnp.sum(stats[:, 0, 0])
    perplexity = stats[4, 1, 0]
    return motion, loss, perplexity
